# baseline (speedup 1.0000x reference)
import jax
import jax.numpy as jnp
from jax import lax
from jax.experimental import pallas as pl
from jax.experimental.pallas import tpu as pltpu
from jax.experimental.pallas import tpu_sc as plsc

_B, _S, _D = 16, 4096, 1024
_L = 16


def _body(x_hbm, m_hbm, out_hbm, mrow):
    wid = lax.axis_index("s")
    pltpu.sync_copy(m_hbm.at[wid], mrow)
    acc = jnp.zeros((_L,), jnp.int32)
    for j in range(_S // 64):
        acc = acc + plsc.bitcast(mrow[pl.ds(j * 64, 64)], jnp.int32)
    t = (acc & 0x00FF00FF) + ((acc >> 8) & 0x00FF00FF)
    s = jnp.sum(t)
    total = (s & 0xFFFF) + (s >> 16)
    idx = jnp.where(total > 0, total - 1, _S - 1)
    pltpu.sync_copy(x_hbm.at[wid, pl.ds(idx, 1)], out_hbm.at[pl.ds(wid, 1)])


def kernel(x, mask):
    m8 = mask.astype(jnp.int8)
    mesh = plsc.VectorSubcoreMesh(
        core_axis_name="c", subcore_axis_name="s", num_cores=1
    )
    run = pl.kernel(
        _body,
        mesh=mesh,
        out_type=jax.ShapeDtypeStruct((_B, _D), jnp.float32),
        scratch_types=[pltpu.VMEM((_S,), jnp.int8)],
        compiler_params=pltpu.CompilerParams(
            needs_layout_passes=False, use_tc_tiling_on_sc=False
        ),
    )
    return run(x, m8)


# speedup vs baseline: 9.5884x; 9.5884x over previous
import jax
import jax.numpy as jnp
from jax import lax
from jax.experimental import pallas as pl
from jax.experimental.pallas import tpu as pltpu
from jax.experimental.pallas import tpu_sc as plsc

_B, _S, _D = 16, 4096, 1024
_L = 16


def _body(x_hbm, m_hbm, out_hbm, mrow):
    wid = lax.axis_index("s")
    pltpu.sync_copy(m_hbm.at[wid], mrow)
    acc = mrow[pl.ds(0, _L)]
    for j in range(1, _S // _L):
        acc = acc + mrow[pl.ds(j * _L, _L)]
    total = jnp.sum(acc)
    idx = jnp.where(total > 0, total - 1, _S - 1)
    pltpu.sync_copy(x_hbm.at[wid, pl.ds(idx, 1)], out_hbm.at[pl.ds(wid, 1)])


def kernel(x, mask):
    m32 = mask.astype(jnp.int32)
    mesh = plsc.VectorSubcoreMesh(
        core_axis_name="c", subcore_axis_name="s", num_cores=1
    )
    run = pl.kernel(
        _body,
        mesh=mesh,
        out_type=jax.ShapeDtypeStruct((_B, _D), jnp.float32),
        scratch_types=[pltpu.VMEM((_S,), jnp.int32)],
        compiler_params=pltpu.CompilerParams(needs_layout_passes=False),
    )
    return run(x, m32)


# speedup vs baseline: 10.7005x; 1.1160x over previous
import jax
import jax.numpy as jnp
from jax import lax
from jax.experimental import pallas as pl
from jax.experimental.pallas import tpu as pltpu
from jax.experimental.pallas import tpu_sc as plsc

_B, _S, _D = 16, 4096, 1024
_L = 16
_UNROLL = 8


def _body(x_hbm, m_hbm, out_hbm, mrow, row):
    wid = lax.axis_index("s")
    pltpu.sync_copy(m_hbm.at[wid], mrow)

    def step(i, a):
        base = i * (_L * _UNROLL)
        for j in range(_UNROLL):
            a = a + mrow[pl.ds(base + j * _L, _L)]
        return a

    acc = lax.fori_loop(
        0, _S // (_L * _UNROLL), step, jnp.zeros((_L,), jnp.int32)
    )
    total = jnp.sum(acc)
    idx = jnp.where(total > 0, total - 1, _S - 1)
    pltpu.sync_copy(x_hbm.at[wid, idx], row)
    pltpu.sync_copy(row, out_hbm.at[wid])


def kernel(x, mask):
    m32 = mask.astype(jnp.int32)
    mesh = plsc.VectorSubcoreMesh(
        core_axis_name="c", subcore_axis_name="s", num_cores=1
    )
    run = pl.kernel(
        _body,
        mesh=mesh,
        out_type=jax.ShapeDtypeStruct((_B, _D), jnp.float32),
        scratch_types=[
            pltpu.VMEM((_S,), jnp.int32),
            pltpu.VMEM((_D,), jnp.float32),
        ],
        compiler_params=pltpu.CompilerParams(
            needs_layout_passes=False,
            disable_bounds_checks=True,
            disable_semaphore_checks=True,
            skip_device_barrier=True,
        ),
    )
    return run(x, m32)


# speedup vs baseline: 11.0176x; 1.0296x over previous
import jax
import jax.numpy as jnp
from jax import lax
from jax.experimental import pallas as pl
from jax.experimental.pallas import tpu as pltpu
from jax.experimental.pallas import tpu_sc as plsc

_B, _S, _D = 16, 4096, 1024
_L = 16
_H = _S // 2
_HD = _D // 2
_UNROLL = 8


def _half_sum(mrow, base, a):
    def step(i, acc):
        off = base + i * (_L * _UNROLL)
        for j in range(_UNROLL):
            acc = acc + mrow[pl.ds(off + j * _L, _L)]
        return acc

    return lax.fori_loop(0, _H // (_L * _UNROLL), step, a)


def _body(x_hbm, m_hbm, out_hbm, mrow, row, s0, s1, s2, s3):
    wid = lax.axis_index("s")
    cm0 = pltpu.async_copy(m_hbm.at[wid, pl.ds(0, _H)], mrow.at[pl.ds(0, _H)], s0)
    cm1 = pltpu.async_copy(m_hbm.at[wid, pl.ds(_H, _H)], mrow.at[pl.ds(_H, _H)], s1)
    cm0.wait()
    acc = _half_sum(mrow, 0, jnp.zeros((_L,), jnp.int32))
    cm1.wait()
    acc = _half_sum(mrow, _H, acc)
    total = jnp.sum(acc)
    idx = jnp.where(total > 0, total - 1, _S - 1)
    r0 = pltpu.async_copy(
        x_hbm.at[wid, idx, pl.ds(0, _HD)], row.at[pl.ds(0, _HD)], s0
    )
    r1 = pltpu.async_copy(
        x_hbm.at[wid, idx, pl.ds(_HD, _HD)], row.at[pl.ds(_HD, _HD)], s1
    )
    r0.wait()
    w0 = pltpu.async_copy(
        row.at[pl.ds(0, _HD)], out_hbm.at[wid, pl.ds(0, _HD)], s2
    )
    r1.wait()
    w1 = pltpu.async_copy(
        row.at[pl.ds(_HD, _HD)], out_hbm.at[wid, pl.ds(_HD, _HD)], s3
    )
    w0.wait()
    w1.wait()


def kernel(x, mask):
    m32 = mask.astype(jnp.int32)
    mesh = plsc.VectorSubcoreMesh(
        core_axis_name="c", subcore_axis_name="s", num_cores=1
    )
    run = pl.kernel(
        _body,
        mesh=mesh,
        out_type=jax.ShapeDtypeStruct((_B, _D), jnp.float32),
        scratch_types=[
            pltpu.VMEM((_S,), jnp.int32),
            pltpu.VMEM((_D,), jnp.float32),
            pltpu.SemaphoreType.DMA,
            pltpu.SemaphoreType.DMA,
            pltpu.SemaphoreType.DMA,
            pltpu.SemaphoreType.DMA,
        ],
        compiler_params=pltpu.CompilerParams(
            needs_layout_passes=False,
            disable_bounds_checks=True,
            disable_semaphore_checks=True,
            skip_device_barrier=True,
        ),
    )
    return run(x, m32)
